# Initial kernel scaffold; baseline (speedup 1.0000x reference)
import jax
import jax.numpy as jnp
from jax.experimental import pallas as pl


def kernel(y1, y2):
    raise NotImplementedError("write your pallas kernel here")


# speedup vs baseline: 2.7762x; 2.7762x over previous
import functools

import jax
import jax.numpy as jnp
from jax import lax
from jax.experimental import pallas as pl
from jax.experimental.pallas import tpu as pltpu

_NORM_EPS = 1e-12


def _pick_block(n: int, cap: int) -> int:
    for b in (cap, cap // 2, cap // 4, cap // 8, 32, 16, 8):
        if b >= 8 and n % b == 0:
            return b
    raise ValueError(f"n={n} must be a multiple of 8")


def _prep_kernel(y1_ref, y2_ref, n1_ref, n2_ref, pos_ref, d11_ref, d22_ref,
                 *, inv_tau: float):
    y1 = y1_ref[...]
    y2 = y2_ref[...]
    r1 = jax.lax.rsqrt(jnp.maximum(jnp.sum(y1 * y1, axis=-1, keepdims=True),
                                   _NORM_EPS * _NORM_EPS))
    r2 = jax.lax.rsqrt(jnp.maximum(jnp.sum(y2 * y2, axis=-1, keepdims=True),
                                   _NORM_EPS * _NORM_EPS))
    n1 = y1 * r1
    n2 = y2 * r2
    n1_ref[...] = n1.astype(jnp.bfloat16)
    n2_ref[...] = n2.astype(jnp.bfloat16)
    pos_ref[...] = jnp.exp(jnp.sum(n1 * n2, axis=-1, keepdims=True) * inv_tau)
    d11_ref[...] = jnp.exp(jnp.sum(n1 * n1, axis=-1, keepdims=True) * inv_tau)
    d22_ref[...] = jnp.exp(jnp.sum(n2 * n2, axis=-1, keepdims=True) * inv_tau)


def _sums_kernel(n1r_ref, n2r_ref, n1c_ref, n2c_ref,
                 s21_ref, p12_ref, p11_ref, p22_ref, acc_ref,
                 *, inv_tau: float):
    j = pl.program_id(1)

    @pl.when(j == 0)
    def _init():
        acc_ref[...] = jnp.zeros_like(acc_ref)

    n1r = n1r_ref[...]
    n2r = n2r_ref[...]
    n1c = n1c_ref[...]
    n2c = n2c_ref[...]

    def expsim(a, b):
        s = lax.dot_general(a, b, (((1,), (1,)), ((), ())),
                            preferred_element_type=jnp.float32)
        return jnp.exp(s * inv_tau)

    e21 = expsim(n2r, n1c)
    acc_ref[...] += jnp.sum(e21, axis=1, keepdims=True)
    p12_ref[...] = jnp.sum(e21, axis=0, keepdims=True)[None]
    e11 = expsim(n1r, n1c)
    p11_ref[...] = jnp.sum(e11, axis=0, keepdims=True)[None]
    e22 = expsim(n2r, n2c)
    p22_ref[...] = jnp.sum(e22, axis=0, keepdims=True)[None]

    @pl.when(j == pl.num_programs(1) - 1)
    def _done():
        s21_ref[...] = acc_ref[...]


def _loss_kernel(p12_ref, p11_ref, p22_ref, s21_ref, pos_ref, d11_ref,
                 d22_ref, out_ref):
    s12 = jnp.sum(p12_ref[...], axis=0)
    s11 = jnp.sum(p11_ref[...], axis=0)
    s22 = jnp.sum(p22_ref[...], axis=0)
    den1 = s12 + s11 - d11_ref[...]
    den2 = s21_ref[...] + s22 - d22_ref[...]
    out_ref[...] = (0.5 * (jnp.log(den1) + jnp.log(den2))
                    - jnp.log(pos_ref[...]))


def _mi_loss(y1, y2, tau: float):
    n, d = y1.shape
    inv_tau = 1.0 / float(tau)
    y1 = jnp.asarray(y1, jnp.float32)
    y2 = jnp.asarray(y2, jnp.float32)

    bm = _pick_block(n, 512)
    p = n // bm

    ba = _pick_block(n, 256)
    vec = jax.ShapeDtypeStruct((n, 1), jnp.float32)
    emb = jax.ShapeDtypeStruct((n, d), jnp.bfloat16)
    row_a = pl.BlockSpec((ba, d), lambda i: (i, 0))
    col_a = pl.BlockSpec((ba, 1), lambda i: (i, 0))
    n1b, n2b, pos, d11, d22 = pl.pallas_call(
        functools.partial(_prep_kernel, inv_tau=inv_tau),
        out_shape=(emb, emb, vec, vec, vec),
        grid=(n // ba,),
        in_specs=[row_a, row_a],
        out_specs=(row_a, row_a, col_a, col_a, col_a),
        compiler_params=pltpu.CompilerParams(
            dimension_semantics=("parallel",)),
    )(y1, y2)

    row_spec = pl.BlockSpec((bm, d), lambda i, j: (i, 0))
    col_spec = pl.BlockSpec((bm, d), lambda i, j: (j, 0))
    part = jax.ShapeDtypeStruct((p, 1, n), jnp.float32)
    part_spec = pl.BlockSpec((1, 1, bm), lambda i, j: (i, 0, j))
    s21, p12, p11, p22 = pl.pallas_call(
        functools.partial(_sums_kernel, inv_tau=inv_tau),
        out_shape=(jax.ShapeDtypeStruct((n, 1), jnp.float32),
                   part, part, part),
        grid=(p, p),
        in_specs=[row_spec, row_spec, col_spec, col_spec],
        out_specs=(pl.BlockSpec((bm, 1), lambda i, j: (i, 0)),
                   part_spec, part_spec, part_spec),
        scratch_shapes=[pltpu.VMEM((bm, 1), jnp.float32)],
        compiler_params=pltpu.CompilerParams(
            dimension_semantics=("parallel", "arbitrary")),
    )(n1b, n2b, n1b, n2b)

    bc = _pick_block(n, 4096)
    lane = lambda v: v.reshape(1, n)
    part_in = pl.BlockSpec((p, 1, bc), lambda r: (0, 0, r))
    lane_in = pl.BlockSpec((1, bc), lambda r: (0, r))
    per_row = pl.pallas_call(
        _loss_kernel,
        out_shape=jax.ShapeDtypeStruct((1, n), jnp.float32),
        grid=(n // bc,),
        in_specs=[part_in, part_in, part_in,
                  lane_in, lane_in, lane_in, lane_in],
        out_specs=lane_in,
        compiler_params=pltpu.CompilerParams(
            dimension_semantics=("parallel",)),
    )(p12, p11, p22, lane(s21), lane(pos), lane(d11), lane(d22))

    return jnp.mean(per_row)


def kernel(y1, y2):
    return _mi_loss(y1, y2, tau=0.5)


# speedup vs baseline: 6.1391x; 2.2113x over previous
import functools

import jax
import jax.numpy as jnp
import numpy as np
from jax import lax
from jax.experimental import pallas as pl
from jax.experimental.pallas import tpu as pltpu

_NORM_EPS = 1e-12


def _pick_block(n: int, cap: int) -> int:
    for b in (cap, cap // 2, cap // 4, cap // 8, 32, 16, 8):
        if b >= 8 and n % b == 0:
            return b
    raise ValueError(f"n={n} must be a multiple of 8")


def _prep_kernel(y1_ref, y2_ref, n1_ref, n2_ref, pos_ref, d11_ref, d22_ref,
                 *, inv_tau: float, mxu_scale: float):
    y1 = y1_ref[...]
    y2 = y2_ref[...]
    sq1 = jnp.sum(y1 * y1, axis=-1, keepdims=True)
    sq2 = jnp.sum(y2 * y2, axis=-1, keepdims=True)
    r1 = jax.lax.rsqrt(jnp.maximum(sq1, _NORM_EPS * _NORM_EPS))
    r2 = jax.lax.rsqrt(jnp.maximum(sq2, _NORM_EPS * _NORM_EPS))
    n1 = y1 * r1
    n2 = y2 * r2
    n1_ref[...] = (n1 * mxu_scale).astype(n1_ref.dtype)
    n2_ref[...] = (n2 * mxu_scale).astype(n2_ref.dtype)
    ba = y1.shape[0]
    pos = jnp.exp(jnp.sum(n1 * n2, axis=-1, keepdims=True) * inv_tau)
    d11 = jnp.exp((sq1 * r1 * r1) * inv_tau)
    d22 = jnp.exp((sq2 * r2 * r2) * inv_tau)
    pos_ref[...] = pos.reshape(1, ba)
    d11_ref[...] = d11.reshape(1, ba)
    d22_ref[...] = d22.reshape(1, ba)


def _sums_kernel(n1r_ref, n2r_ref, n1c_ref, n2c_ref,
                 s21_ref, s11_ref, s22_ref, p12_ref, p11_ref, p22_ref,
                 a21_ref, a11_ref, a22_ref, *, nblk: int, ratio: int):
    pp = pl.program_id(0)
    j = pl.program_id(1)
    i = jnp.where(pp % 2 == 0, pp // 2, nblk - 1 - pp // 2)

    @pl.when(j == 0)
    def _init():
        a21_ref[...] = jnp.zeros_like(a21_ref)
        a11_ref[...] = jnp.zeros_like(a11_ref)
        a22_ref[...] = jnp.zeros_like(a22_ref)

    n1r = n1r_ref[...]
    n2r = n2r_ref[...]
    n1c = n1c_ref[...]
    n2c = n2c_ref[...]

    def expsim(a, b):
        s = lax.dot_general(a, b, (((1,), (1,)), ((), ())),
                            preferred_element_type=jnp.float32)
        return jnp.exp2(s)

    def rowsum(e):
        return jnp.sum(e, axis=1, keepdims=True, dtype=jnp.float32)

    def colsum(e):
        return jnp.sum(e, axis=0, keepdims=True, dtype=jnp.float32)[None]

    e21 = expsim(n2r, n1c)
    a21_ref[...] += rowsum(e21)
    p12_ref[...] = colsum(e21)

    @pl.when(j >= ratio * i)
    def _sym():
        up = (j >= ratio * (i + 1)).astype(jnp.float32)
        e11 = expsim(n1r, n1c)
        a11_ref[...] += rowsum(e11)
        p11_ref[...] = colsum(e11) * up
        e22 = expsim(n2r, n2c)
        a22_ref[...] += rowsum(e22)
        p22_ref[...] = colsum(e22) * up

    @pl.when(j < ratio * i)
    def _skip():
        p11_ref[...] = jnp.zeros_like(p11_ref)
        p22_ref[...] = jnp.zeros_like(p22_ref)

    @pl.when(j == pl.num_programs(1) - 1)
    def _done():
        bm = a21_ref.shape[0]
        s21_ref[...] = a21_ref[...].reshape(1, bm)
        s11_ref[...] = a11_ref[...].reshape(1, bm)
        s22_ref[...] = a22_ref[...].reshape(1, bm)


def _loss_kernel(p12_ref, p11_ref, p22_ref, s21_ref, s11r_ref, s22r_ref,
                 pos_ref, d11_ref, d22_ref, out_ref):
    s12 = jnp.sum(p12_ref[...], axis=0)
    s11 = jnp.sum(p11_ref[...], axis=0) + s11r_ref[...]
    s22 = jnp.sum(p22_ref[...], axis=0) + s22r_ref[...]
    den1 = s12 + s11 - d11_ref[...]
    den2 = s21_ref[...] + s22 - d22_ref[...]
    per_row = (0.5 * (jnp.log(den1) + jnp.log(den2))
               - jnp.log(pos_ref[...]))
    out_ref[0, 0] = jnp.sum(per_row) / per_row.shape[1]


def _mi_loss(y1, y2, tau: float):
    n, d = y1.shape
    inv_tau = 1.0 / float(tau)
    y1 = jnp.asarray(y1, jnp.float32)
    y2 = jnp.asarray(y2, jnp.float32)

    bm = _pick_block(n, 2048)
    bn = _pick_block(n, 2048)
    p = n // bm
    pc = n // bn
    ratio = bm // bn
    mxu_scale = float(np.sqrt(inv_tau * np.log2(np.e)))

    ba = _pick_block(n, 1024)
    vec = jax.ShapeDtypeStruct((1, n), jnp.float32)
    emb = jax.ShapeDtypeStruct((n, d), jnp.float8_e4m3fn)
    row_a = pl.BlockSpec((ba, d), lambda i: (i, 0))
    lane_a = pl.BlockSpec((1, ba), lambda i: (0, i))
    n1b, n2b, pos, d11, d22 = pl.pallas_call(
        functools.partial(_prep_kernel, inv_tau=inv_tau, mxu_scale=mxu_scale),
        out_shape=(emb, emb, vec, vec, vec),
        grid=(n // ba,),
        in_specs=[row_a, row_a],
        out_specs=(row_a, row_a, lane_a, lane_a, lane_a),
        compiler_params=pltpu.CompilerParams(
            dimension_semantics=("parallel",)),
    )(y1, y2)

    def _perm(pp):
        return jnp.where(pp % 2 == 0, pp // 2, p - 1 - pp // 2)

    row_spec = pl.BlockSpec((bm, d), lambda pp, j: (_perm(pp), 0))
    col_spec = pl.BlockSpec((bn, d), lambda pp, j: (j, 0))
    rvec_spec = pl.BlockSpec((1, bm), lambda pp, j: (0, _perm(pp)))
    rvec = jax.ShapeDtypeStruct((1, n), jnp.float32)
    part = jax.ShapeDtypeStruct((p, 1, n), jnp.float32)
    part_spec = pl.BlockSpec((1, 1, bn), lambda pp, j: (pp, 0, j))
    s21, s11r, s22r, p12, p11, p22 = pl.pallas_call(
        functools.partial(_sums_kernel, nblk=p, ratio=ratio),
        out_shape=(rvec, rvec, rvec, part, part, part),
        grid=(p, pc),
        in_specs=[row_spec, row_spec, col_spec, col_spec],
        out_specs=(rvec_spec, rvec_spec, rvec_spec,
                   part_spec, part_spec, part_spec),
        scratch_shapes=[pltpu.VMEM((bm, 1), jnp.float32)] * 3,
        compiler_params=pltpu.CompilerParams(
            dimension_semantics=("parallel", "arbitrary")),
    )(n1b, n2b, n1b, n2b)

    part_in = pl.BlockSpec((p, 1, n), lambda: (0, 0, 0))
    lane_in = pl.BlockSpec((1, n), lambda: (0, 0))
    loss = pl.pallas_call(
        _loss_kernel,
        out_shape=jax.ShapeDtypeStruct((1, 1), jnp.float32),
        in_specs=[part_in, part_in, part_in,
                  lane_in, lane_in, lane_in, lane_in, lane_in, lane_in],
        out_specs=pl.BlockSpec(memory_space=pltpu.SMEM),
    )(p12, p11, p22, s21, s11r, s22r, pos, d11, d22)

    return loss[0, 0]


def kernel(y1, y2):
    return _mi_loss(y1, y2, tau=0.5)
